# baseline (speedup 1.0000x reference)
import functools

import numpy as np
import jax
import jax.numpy as jnp
from jax import lax
from jax.experimental import pallas as pl
from jax.experimental.pallas import tpu as pltpu
from jax.experimental.pallas import tpu_sc as plsc

_N, _D = 16384, 512
_NUM_IV = 128


def _intervention_idx():
    perm = jax.random.permutation(jax.random.key(42), _D)
    return perm[:_NUM_IV].astype(jnp.int32)

_NC, _NS, _L = 2, 16, 16
_NW = _NC * _NS
_ROWS_W = _N // _NW
_CHUNK = 32
_NCHUNK = _ROWS_W // _CHUNK
_G = _NUM_IV // _L

_mesh = plsc.VectorSubcoreMesh(core_axis_name="c", subcore_axis_name="s")


@functools.partial(
    pl.kernel,
    out_type=jax.ShapeDtypeStruct((_N, _D), jnp.float32),
    mesh=_mesh,
    scratch_types=[
        pltpu.VMEM((_CHUNK, _D), jnp.float32),
        pltpu.VMEM((_CHUNK, _D), jnp.float32),
        pltpu.VMEM((_NUM_IV,), jnp.int32),
    ],
    compiler_params=pltpu.CompilerParams(
        use_tc_tiling_on_sc=False, needs_layout_passes=False
    ),
)
def _sc_intervene(x_hbm, c_hbm, idx_hbm, out_hbm, ob, cb, idxb):
    wid = lax.axis_index("s") * _NC + lax.axis_index("c")
    base = wid * _ROWS_W
    pltpu.sync_copy(idx_hbm, idxb)
    colvs = [idxb[pl.ds(g * _L, _L)] for g in range(_G)]

    def chunk_body(k, carry):
        r0 = base + k * _CHUNK
        pltpu.sync_copy(x_hbm.at[pl.ds(r0, _CHUNK)], ob)
        pltpu.sync_copy(c_hbm.at[pl.ds(r0, _CHUNK)], cb)

        def row_body(r, c):
            rv = jnp.full((_L,), r, jnp.int32)
            for g in range(_G):
                v = plsc.load_gather(cb, [rv, colvs[g]])
                plsc.store_scatter(ob, [rv, colvs[g]], v)
            return c

        lax.fori_loop(0, _CHUNK, row_body, 0)
        pltpu.sync_copy(ob, out_hbm.at[pl.ds(r0, _CHUNK)])
        return carry

    lax.fori_loop(0, _NCHUNK, chunk_body, 0)


def kernel(x, concepts):
    return _sc_intervene(x, concepts, _intervention_idx())


# speedup vs baseline: 2.7593x; 2.7593x over previous
import functools

import jax
import jax.numpy as jnp
from jax import lax
from jax.experimental import pallas as pl
from jax.experimental.pallas import tpu as pltpu
from jax.experimental.pallas import tpu_sc as plsc

_N, _D = 16384, 512
_NUM_IV = 128
_NC, _NS, _L = 2, 16, 16
_NW = _NC * _NS
_ROWS_W = _N // _NW
_CHUNK = 32
_NCHUNK = _ROWS_W // _CHUNK
_G = _NUM_IV // _L

_mesh = plsc.VectorSubcoreMesh(core_axis_name="c", subcore_axis_name="s")


@functools.partial(
    pl.kernel,
    out_type=jax.ShapeDtypeStruct((_N, _D), jnp.float32),
    mesh=_mesh,
    scratch_types=[
        pltpu.VMEM((_CHUNK, _D), jnp.float32),
        pltpu.VMEM((_CHUNK, _D), jnp.float32),
        pltpu.VMEM((_CHUNK, _D), jnp.float32),
        pltpu.VMEM((_CHUNK, _D), jnp.float32),
        pltpu.VMEM((_NUM_IV,), jnp.int32),
        pltpu.SemaphoreType.DMA,
        pltpu.SemaphoreType.DMA,
        pltpu.SemaphoreType.DMA,
        pltpu.SemaphoreType.DMA,
    ],
    compiler_params=pltpu.CompilerParams(
        use_tc_tiling_on_sc=True, needs_layout_passes=False
    ),
)
def _sc_intervene(x_hbm, c_hbm, idx_hbm, out_hbm, ob0, ob1, cb0, cb1, idxb,
                  si0, si1, so0, so1):
    wid = lax.axis_index("s") * _NC + lax.axis_index("c")
    base = wid * _ROWS_W
    pltpu.sync_copy(idx_hbm, idxb)
    colvs = [idxb[pl.ds(g * _L, _L)] for g in range(_G)]

    obs, cbs, sis, sos = (ob0, ob1), (cb0, cb1), (si0, si1), (so0, so1)

    def in_copies(k, s):
        r0 = base + k * _CHUNK
        return (
            pltpu.make_async_copy(x_hbm.at[pl.ds(r0, _CHUNK)], obs[s], sis[s]),
            pltpu.make_async_copy(c_hbm.at[pl.ds(r0, _CHUNK)], cbs[s], sis[s]),
        )

    def out_copy(k, s):
        r0 = base + k * _CHUNK
        return pltpu.make_async_copy(obs[s], out_hbm.at[pl.ds(r0, _CHUNK)], sos[s])

    def compute(s):
        ob, cb = obs[s], cbs[s]

        def row_body(r, c):
            rv = jnp.full((_L,), r, jnp.int32)
            for g in range(_G):
                v = plsc.load_gather(cb, [rv, colvs[g]])
                plsc.store_scatter(ob, [rv, colvs[g]], v)
            return c

        lax.fori_loop(0, _CHUNK, row_body, 0)

    for c in in_copies(0, 0):
        c.start()
    for k in range(_NCHUNK):
        s = k % 2
        for c in in_copies(k, s):
            c.wait()
        if k > 0:
            out_copy(k - 1, 1 - s).wait()
        if k + 1 < _NCHUNK:
            for c in in_copies(k + 1, 1 - s):
                c.start()
        compute(s)
        out_copy(k, s).start()
    out_copy(_NCHUNK - 1, (_NCHUNK - 1) % 2).wait()


def _intervention_idx():
    perm = jax.random.permutation(jax.random.key(42), _D)
    return perm[:_NUM_IV].astype(jnp.int32)


def kernel(x, concepts):
    return _sc_intervene(x, concepts, _intervention_idx())


# speedup vs baseline: 2.8411x; 1.0297x over previous
import functools

import jax
import jax.numpy as jnp
from jax import lax
from jax.experimental import pallas as pl
from jax.experimental.pallas import tpu as pltpu
from jax.experimental.pallas import tpu_sc as plsc

_N, _D = 16384, 512
_NUM_IV = 128
_NC, _NS, _L = 2, 16, 16
_NW = _NC * _NS
_ROWS_W = _N // _NW
_CHUNK = 32
_NCHUNK = _ROWS_W // _CHUNK
_G = _NUM_IV // _L

_mesh = plsc.VectorSubcoreMesh(core_axis_name="c", subcore_axis_name="s")


@functools.partial(
    pl.kernel,
    out_type=jax.ShapeDtypeStruct((_N, _D), jnp.float32),
    mesh=_mesh,
    scratch_types=[
        pltpu.VMEM((_CHUNK, _D), jnp.float32),
        pltpu.VMEM((_CHUNK, _D), jnp.float32),
        pltpu.VMEM((_CHUNK, _D), jnp.float32),
        pltpu.VMEM((_CHUNK, _D), jnp.float32),
        pltpu.VMEM((_CHUNK, _D), jnp.float32),
        pltpu.VMEM((_CHUNK, _D), jnp.float32),
        pltpu.VMEM((_NUM_IV,), jnp.int32),
        pltpu.SemaphoreType.DMA,
        pltpu.SemaphoreType.DMA,
        pltpu.SemaphoreType.DMA,
        pltpu.SemaphoreType.DMA,
        pltpu.SemaphoreType.DMA,
        pltpu.SemaphoreType.DMA,
    ],
    compiler_params=pltpu.CompilerParams(
        use_tc_tiling_on_sc=True, needs_layout_passes=False
    ),
)
def _sc_intervene(x_hbm, c_hbm, idx_hbm, out_hbm, ob0, ob1, ob2,
                  cb0, cb1, cb2, idxb, si0, si1, si2, so0, so1, so2):
    wid = lax.axis_index("s") * _NC + lax.axis_index("c")
    base = wid * _ROWS_W
    pltpu.sync_copy(idx_hbm, idxb)
    colvs = [idxb[pl.ds(g * _L, _L)] for g in range(_G)]

    obs, cbs = (ob0, ob1, ob2), (cb0, cb1, cb2)
    sis, sos = (si0, si1, si2), (so0, so1, so2)

    def in_copies(k, s):
        r0 = base + k * _CHUNK
        return (
            pltpu.make_async_copy(x_hbm.at[pl.ds(r0, _CHUNK)], obs[s], sis[s]),
            pltpu.make_async_copy(c_hbm.at[pl.ds(r0, _CHUNK)], cbs[s], sis[s]),
        )

    def out_copy(k, s):
        r0 = base + k * _CHUNK
        return pltpu.make_async_copy(obs[s], out_hbm.at[pl.ds(r0, _CHUNK)], sos[s])

    def compute(s):
        ob, cb = obs[s], cbs[s]

        def row_body(r, c):
            rv = jnp.full((_L,), r, jnp.int32)
            for g in range(_G):
                v = plsc.load_gather(cb, [rv, colvs[g]])
                plsc.store_scatter(ob, [rv, colvs[g]], v)
            return c

        lax.fori_loop(0, _CHUNK, row_body, 0)

    for kk in (0, 1):
        for c in in_copies(kk, kk % 3):
            c.start()
    for k in range(_NCHUNK):
        s = k % 3
        for c in in_copies(k, s):
            c.wait()
        compute(s)
        out_copy(k, s).start()
        if k + 2 < _NCHUNK:
            if k >= 1:
                out_copy(k - 1, (k - 1) % 3).wait()
            for c in in_copies(k + 2, (k + 2) % 3):
                c.start()
    for kk in (_NCHUNK - 3, _NCHUNK - 2, _NCHUNK - 1):
        out_copy(kk, kk % 3).wait()


def _intervention_idx():
    perm = jax.random.permutation(jax.random.key(42), _D)
    return perm[:_NUM_IV].astype(jnp.int32)


def kernel(x, concepts):
    return _sc_intervene(x, concepts, _intervention_idx())
